# initial kernel scaffold (unmeasured)
import jax
import jax.numpy as jnp
from jax import lax
from jax.experimental import pallas as pl
from jax.experimental.pallas import tpu as pltpu

N_DEV = 4
SQ = 2048
SKV = 2048
HQ = 32
DH = 128
D_MODEL = 1024
H_PER = HQ // N_DEV
BLK = 64
QB = 256
N_QB = SQ // QB
SCALE = 0.08838834764831843


def _ag_body(w_ref, out_ref, send_sems, recv_sems):
    my = lax.axis_index("i")
    left = (my + N_DEV - 1) % N_DEV
    right = (my + 1) % N_DEV

    barrier_sem = pltpu.get_barrier_semaphore()
    for nbr in (left, right):
        pl.semaphore_signal(
            barrier_sem, inc=1,
            device_id=(nbr,), device_id_type=pl.DeviceIdType.MESH,
        )
    pl.semaphore_wait(barrier_sem, 2)

    out_ref[pl.ds(my, 1)] = w_ref[...].reshape(1, 2, D_MODEL, D_MODEL)

    for h in range(N_DEV - 1):
        src = (my + N_DEV - h) % N_DEV
        rdma = pltpu.make_async_remote_copy(
            src_ref=out_ref.at[pl.ds(src, 1)],
            dst_ref=out_ref.at[pl.ds(src, 1)],
            send_sem=send_sems.at[h],
            recv_sem=recv_sems.at[h],
            device_id=(right,),
            device_id_type=pl.DeviceIdType.MESH,
        )
        rdma.start()
        rdma.wait()


def _allgather_weights(w_stack):
    return pl.pallas_call(
        _ag_body,
        out_shape=jax.ShapeDtypeStruct((N_DEV, 2, D_MODEL, D_MODEL), jnp.float32),
        in_specs=[pl.BlockSpec(memory_space=pltpu.VMEM)],
        out_specs=pl.BlockSpec(memory_space=pltpu.VMEM),
        scratch_shapes=[
            pltpu.SemaphoreType.DMA((N_DEV - 1,)),
            pltpu.SemaphoreType.DMA((N_DEV - 1,)),
        ],
        compiler_params=pltpu.CompilerParams(collective_id=0),
    )(w_stack)


def _attn_body(my_ref, x_ref, wq_ref, wo_ref, k_ref, v_ref, out_ref):
    h = pl.program_id(0)
    wq = wq_ref[0, 0]
    wo = wo_ref[0, 0]
    k = k_ref[0, :, 0, :]
    v = v_ref[0, :, 0, :]

    rows = lax.broadcasted_iota(jnp.int32, (QB, QB), 0)
    cols = lax.broadcasted_iota(jnp.int32, (QB, QB), 1)
    dmask = (cols // BLK) <= (rows // BLK)

    for qi in range(N_QB):
        L = QB * (qi + 1)
        xblk = x_ref[qi * QB:(qi + 1) * QB, :]
        q = jnp.dot(xblk, wq, preferred_element_type=jnp.float32)
        s = lax.dot_general(
            q, k[:L, :], (((1,), (1,)), ((), ())),
            preferred_element_type=jnp.float32,
        ) * SCALE
        sd = jnp.where(dmask, s[:, qi * QB:], -1e9)
        if qi > 0:
            s = jnp.concatenate([s[:, :qi * QB], sd], axis=1)
        else:
            s = sd
        m = jnp.max(s, axis=1, keepdims=True)
        w = jnp.exp(s - m)
        denom = jnp.sum(w, axis=1, keepdims=True)
        ctx = lax.dot_general(
            w, v[:L, :], (((1,), (0,)), ((), ())),
            preferred_element_type=jnp.float32,
        ) / denom
        contrib = jnp.dot(ctx, wo, preferred_element_type=jnp.float32)

        @pl.when(h == 0)
        def _(contrib=contrib, qi=qi):
            out_ref[qi * QB:(qi + 1) * QB, :] = contrib

        @pl.when(h > 0)
        def _(contrib=contrib, qi=qi):
            out_ref[qi * QB:(qi + 1) * QB, :] += contrib


def _attention(my_arr, x2d, gathered, K_ext, V_ext):
    grid_spec = pltpu.PrefetchScalarGridSpec(
        num_scalar_prefetch=1,
        grid=(HQ,),
        in_specs=[
            pl.BlockSpec((SQ, D_MODEL), lambda h, my: (0, 0)),
            pl.BlockSpec((1, 1, D_MODEL, DH), lambda h, my: (h // H_PER, 0, 0, h % H_PER)),
            pl.BlockSpec((1, 1, DH, D_MODEL), lambda h, my: (h // H_PER, 1, h % H_PER, 0)),
            pl.BlockSpec((1, SKV, 1, DH), lambda h, my: (my[0], 0, h, 0)),
            pl.BlockSpec((1, SKV, 1, DH), lambda h, my: (my[0], 0, h, 0)),
        ],
        out_specs=pl.BlockSpec((SQ, D_MODEL), lambda h, my: (0, 0)),
    )
    return pl.pallas_call(
        _attn_body,
        grid_spec=grid_spec,
        out_shape=jax.ShapeDtypeStruct((SQ, D_MODEL), jnp.float32),
    )(my_arr, x2d, gathered, gathered, K_ext, V_ext)


def kernel(x, Wq, K_ext, V_ext, Wo):
    my = lax.axis_index("i")
    w_stack = jnp.stack([Wq, Wo])
    gathered = _allgather_weights(w_stack)
    my_arr = jnp.reshape(my, (1,)).astype(jnp.int32)
    out = _attention(my_arr, x[0], gathered, K_ext, V_ext)
    return out[None]


# baseline (device time: 1000952 ns/iter reference)
import jax
import jax.numpy as jnp
from jax import lax
from jax.experimental import pallas as pl
from jax.experimental.pallas import tpu as pltpu

N_DEV = 4
SQ = 2048
SKV = 2048
HQ = 32
DH = 128
D_MODEL = 1024
H_PER = HQ // N_DEV
BLK = 64
QB = 256
N_QB = SQ // QB
SCALE = 0.08838834764831843


def _ag_body(w_ref, out_ref, send_sems, recv_sems):
    my = lax.axis_index("i")
    left = (my + N_DEV - 1) % N_DEV
    right = (my + 1) % N_DEV

    barrier_sem = pltpu.get_barrier_semaphore()
    for nbr in (left, right):
        pl.semaphore_signal(
            barrier_sem, inc=1,
            device_id=(nbr,), device_id_type=pl.DeviceIdType.MESH,
        )
    pl.semaphore_wait(barrier_sem, 2)

    out_ref[pl.ds(my, 1)] = w_ref[...].reshape(1, 2, D_MODEL, D_MODEL)

    for h in range(N_DEV - 1):
        src = (my + N_DEV - h) % N_DEV
        rdma = pltpu.make_async_remote_copy(
            src_ref=out_ref.at[pl.ds(src, 1)],
            dst_ref=out_ref.at[pl.ds(src, 1)],
            send_sem=send_sems.at[h],
            recv_sem=recv_sems.at[h],
            device_id=(right,),
            device_id_type=pl.DeviceIdType.MESH,
        )
        rdma.start()
        rdma.wait()


def _allgather_weights(w_stack):
    return pl.pallas_call(
        _ag_body,
        out_shape=jax.ShapeDtypeStruct((N_DEV, 2, D_MODEL, D_MODEL), jnp.float32),
        in_specs=[pl.BlockSpec(memory_space=pltpu.VMEM)],
        out_specs=pl.BlockSpec(memory_space=pltpu.VMEM),
        scratch_shapes=[
            pltpu.SemaphoreType.DMA((N_DEV - 1,)),
            pltpu.SemaphoreType.DMA((N_DEV - 1,)),
        ],
        compiler_params=pltpu.CompilerParams(collective_id=0),
    )(w_stack)


def _attn_body(my_ref, x_ref, wq_ref, wo_ref, k_ref, v_ref, out_ref):
    h = pl.program_id(0)
    wq = wq_ref[0, 0]
    wo = wo_ref[0, 0]
    k = k_ref[0]
    v = v_ref[0]

    rows = lax.broadcasted_iota(jnp.int32, (QB, QB), 0)
    cols = lax.broadcasted_iota(jnp.int32, (QB, QB), 1)
    dmask = (cols // BLK) <= (rows // BLK)

    for qi in range(N_QB):
        L = QB * (qi + 1)
        xblk = x_ref[qi * QB:(qi + 1) * QB, :]
        q = jnp.dot(xblk, wq, preferred_element_type=jnp.float32)
        s = lax.dot_general(
            q, k[:L, :], (((1,), (1,)), ((), ())),
            preferred_element_type=jnp.float32,
        ) * SCALE
        sd = jnp.where(dmask, s[:, qi * QB:], -1e9)
        if qi > 0:
            s = jnp.concatenate([s[:, :qi * QB], sd], axis=1)
        else:
            s = sd
        m = jnp.max(s, axis=1, keepdims=True)
        w = jnp.exp(s - m)
        denom = jnp.sum(w, axis=1, keepdims=True)
        ctx = lax.dot_general(
            w, v[:L, :], (((1,), (0,)), ((), ())),
            preferred_element_type=jnp.float32,
        ) / denom
        contrib = jnp.dot(ctx, wo, preferred_element_type=jnp.float32)

        @pl.when(h == 0)
        def _(contrib=contrib, qi=qi):
            out_ref[qi * QB:(qi + 1) * QB, :] = contrib

        @pl.when(h > 0)
        def _(contrib=contrib, qi=qi):
            out_ref[qi * QB:(qi + 1) * QB, :] += contrib


def _attention(my_arr, x2d, gathered, K_ext, V_ext):
    grid_spec = pltpu.PrefetchScalarGridSpec(
        num_scalar_prefetch=1,
        grid=(HQ,),
        in_specs=[
            pl.BlockSpec((SQ, D_MODEL), lambda h, my: (0, 0)),
            pl.BlockSpec((1, 1, D_MODEL, DH), lambda h, my: (h // H_PER, 0, 0, h % H_PER)),
            pl.BlockSpec((1, 1, DH, D_MODEL), lambda h, my: (h // H_PER, 1, h % H_PER, 0)),
            pl.BlockSpec((1, SKV, DH), lambda h, my: (my[0], 0, h)),
            pl.BlockSpec((1, SKV, DH), lambda h, my: (my[0], 0, h)),
        ],
        out_specs=pl.BlockSpec((SQ, D_MODEL), lambda h, my: (0, 0)),
    )
    return pl.pallas_call(
        _attn_body,
        grid_spec=grid_spec,
        out_shape=jax.ShapeDtypeStruct((SQ, D_MODEL), jnp.float32),
    )(my_arr, x2d, gathered, gathered, K_ext, V_ext)


def kernel(x, Wq, K_ext, V_ext, Wo):
    my = lax.axis_index("i")
    w_stack = jnp.stack([Wq, Wo])
    gathered = _allgather_weights(w_stack)
    my_arr = jnp.reshape(my, (1,)).astype(jnp.int32)
    K2 = K_ext.reshape(N_DEV, SKV, HQ * DH)
    V2 = V_ext.reshape(N_DEV, SKV, HQ * DH)
    out = _attention(my_arr, x[0], gathered, K2, V2)
    return out[None]


# device time: 425546 ns/iter; 2.3522x vs baseline; 2.3522x over previous
import jax
import jax.numpy as jnp
from jax import lax
from jax.experimental import pallas as pl
from jax.experimental.pallas import tpu as pltpu

N_DEV = 4
SQ = 2048
SKV = 2048
HQ = 32
DH = 128
D_MODEL = 1024
H_PER = HQ // N_DEV
BLK = 64
QB = 256
N_QB = SQ // QB
SCALE = 0.08838834764831843


def _body(my_ref, x_ref, w_ref, k_hbm, v_hbm, out_ref,
          w_scr, k_scr, v_scr, send_sems, recv_sems, local_sems):
    g = pl.program_id(0)
    my = my_ref[0]
    right = (my + 1) % N_DEV
    left = (my + N_DEV - 1) % N_DEV
    slot = (g + N_DEV - 1) % N_DEV
    grp = (my + N_DEV - g) % N_DEV

    @pl.when(g == 0)
    def _():
        barrier_sem = pltpu.get_barrier_semaphore()
        for nbr in (left, right):
            pl.semaphore_signal(
                barrier_sem, inc=1,
                device_id=(nbr,), device_id_type=pl.DeviceIdType.MESH,
            )
        pl.semaphore_wait(barrier_sem, 2)
        cp = pltpu.make_async_copy(w_ref, w_scr.at[3], local_sems.at[2])
        cp.start()
        cp.wait()
        rdma = pltpu.make_async_remote_copy(
            src_ref=w_scr.at[pl.ds(3, 1)],
            dst_ref=w_scr.at[pl.ds(0, 1)],
            send_sem=send_sems.at[0],
            recv_sem=recv_sems.at[0],
            device_id=(right,),
            device_id_type=pl.DeviceIdType.MESH,
        )
        rdma.start()

    @pl.when(g > 0)
    def _():
        recv = pltpu.make_async_remote_copy(
            src_ref=w_scr.at[pl.ds(slot, 1)],
            dst_ref=w_scr.at[pl.ds(slot, 1)],
            send_sem=send_sems.at[g - 1],
            recv_sem=recv_sems.at[g - 1],
            device_id=(left,),
            device_id_type=pl.DeviceIdType.MESH,
        )
        recv.wait_recv()

    @pl.when((g > 0) & (g < N_DEV - 1))
    def _():
        fwd = pltpu.make_async_remote_copy(
            src_ref=w_scr.at[pl.ds(slot, 1)],
            dst_ref=w_scr.at[pl.ds(g, 1)],
            send_sem=send_sems.at[g],
            recv_sem=recv_sems.at[g],
            device_id=(right,),
            device_id_type=pl.DeviceIdType.MESH,
        )
        fwd.start()

    @pl.when(g == N_DEV - 1)
    def _():
        for h in range(N_DEV - 1):
            s = pltpu.make_async_remote_copy(
                src_ref=w_scr.at[pl.ds((h + N_DEV - 1) % N_DEV, 1)],
                dst_ref=w_scr.at[pl.ds(h, 1)],
                send_sem=send_sems.at[h],
                recv_sem=recv_sems.at[h],
                device_id=(right,),
                device_id_type=pl.DeviceIdType.MESH,
            )
            s.wait_send()

    kcp = pltpu.make_async_copy(
        k_hbm.at[pl.ds(my, 1), :, pl.ds(grp * H_PER, H_PER), :],
        k_scr, local_sems.at[0])
    vcp = pltpu.make_async_copy(
        v_hbm.at[pl.ds(my, 1), :, pl.ds(grp * H_PER, H_PER), :],
        v_scr, local_sems.at[1])
    kcp.start()
    vcp.start()
    kcp.wait()
    vcp.wait()

    wq = w_scr[pl.ds(slot, 1), 0][0]
    wo = w_scr[pl.ds(slot, 1), 1][0]

    rows = lax.broadcasted_iota(jnp.int32, (QB, QB), 0)
    cols = lax.broadcasted_iota(jnp.int32, (QB, QB), 1)
    dmask = (cols // BLK) <= (rows // BLK)

    for qi in range(N_QB):
        L = QB * (qi + 1)
        xblk = x_ref[qi * QB:(qi + 1) * QB, :]
        qall = jnp.dot(xblk, wq, preferred_element_type=jnp.float32)
        ctxs = []
        for h in range(H_PER):
            q = qall[:, h * DH:(h + 1) * DH]
            s = lax.dot_general(
                q, k_scr[0, :L, h, :], (((1,), (1,)), ((), ())),
                preferred_element_type=jnp.float32,
            ) * SCALE
            sd = jnp.where(dmask, s[:, qi * QB:], -1e9)
            if qi > 0:
                s = jnp.concatenate([s[:, :qi * QB], sd], axis=1)
            else:
                s = sd
            m = jnp.max(s, axis=1, keepdims=True)
            w = jnp.exp(s - m)
            denom = jnp.sum(w, axis=1, keepdims=True)
            ctx = lax.dot_general(
                w, v_scr[0, :L, h, :], (((1,), (0,)), ((), ())),
                preferred_element_type=jnp.float32,
            ) / denom
            ctxs.append(ctx)
        ctx_g = jnp.concatenate(ctxs, axis=1).astype(jnp.bfloat16)
        contrib = jnp.dot(ctx_g, wo, preferred_element_type=jnp.float32)

        @pl.when(g == 0)
        def _(contrib=contrib, qi=qi):
            out_ref[qi * QB:(qi + 1) * QB, :] = contrib

        @pl.when(g > 0)
        def _(contrib=contrib, qi=qi):
            out_ref[qi * QB:(qi + 1) * QB, :] += contrib


def kernel(x, Wq, K_ext, V_ext, Wo):
    my = lax.axis_index("i")
    my_arr = jnp.reshape(my, (1,)).astype(jnp.int32)
    w_stack = jnp.stack([Wq, Wo]).astype(jnp.bfloat16)
    x_bf = x[0].astype(jnp.bfloat16)

    grid_spec = pltpu.PrefetchScalarGridSpec(
        num_scalar_prefetch=1,
        grid=(N_DEV,),
        in_specs=[
            pl.BlockSpec((SQ, D_MODEL), lambda g, my: (0, 0)),
            pl.BlockSpec(memory_space=pl.ANY),
            pl.BlockSpec(memory_space=pl.ANY),
            pl.BlockSpec(memory_space=pl.ANY),
        ],
        out_specs=pl.BlockSpec((SQ, D_MODEL), lambda g, my: (0, 0)),
        scratch_shapes=[
            pltpu.VMEM((N_DEV, 2, D_MODEL, D_MODEL), jnp.bfloat16),
            pltpu.VMEM((1, SKV, H_PER, DH), jnp.float32),
            pltpu.VMEM((1, SKV, H_PER, DH), jnp.float32),
            pltpu.SemaphoreType.DMA((N_DEV - 1,)),
            pltpu.SemaphoreType.DMA((N_DEV - 1,)),
            pltpu.SemaphoreType.DMA((3,)),
        ],
    )
    out = pl.pallas_call(
        _body,
        grid_spec=grid_spec,
        out_shape=jax.ShapeDtypeStruct((SQ, D_MODEL), jnp.float32),
        compiler_params=pltpu.CompilerParams(
            collective_id=0,
            vmem_limit_bytes=64 * 1024 * 1024,
        ),
    )(my_arr, x_bf, w_stack, K_ext, V_ext)
    return out[None]
